# Initial kernel scaffold; baseline (speedup 1.0000x reference)
import jax
import jax.numpy as jnp
from jax.experimental import pallas as pl


def kernel(features, edge_index, y, W1, b1, W2, b2, Wy, by, Wfc, bfc):
    raise NotImplementedError("write your pallas kernel here")


# speedup vs baseline: 3.9053x; 3.9053x over previous
import functools

import jax
import jax.numpy as jnp
from jax import lax
from jax.experimental import pallas as pl
from jax.experimental.pallas import tpu as pltpu
from jax.experimental.pallas import tpu_sc as plsc

_N = 10000
_E = 320000
_D = 128
_NCLS = 16

_NC = 2
_NS = 16
_NW = _NC * _NS
_CHUNK = 128
_CPT = 80
_GRP = 8
_NGRP = _CPT // _GRP
_EPAD = _NW * _CPT * _CHUNK
_ROWS = 10112
_ZROWS = _ROWS // _NS


def _sc_propagate_body(sup_hbm, src_hbm, dst_hbm, out_hbm,
                       src_v, dst_v, rows_v, agg_sh,
                       sem0, sem1, semis, semid):
    cid = lax.axis_index("c")
    sid = lax.axis_index("s")
    wid = cid * _NS + sid
    sems = (sem0, sem1)

    @pl.loop(0, _CHUNK)
    def _zero_buf(i):
        for j in range(_D // 16):
            rows_v[0, i, pl.ds(j * 16, 16)] = jnp.zeros((16,), jnp.float32)

    zbase = sid * _ZROWS
    off = 0
    for blk in (128, 128, 128, 128, _ZROWS - 4 * 128):
        pltpu.sync_copy(rows_v.at[0, pl.ds(0, blk)],
                        agg_sh.at[pl.ds(zbase + off, blk)])
        off += blk
    plsc.subcore_barrier()

    ebase = wid * _CPT

    def idx_copies(grp, buf):
        s = pltpu.make_async_copy(
            src_hbm.at[pl.ds(ebase + grp * _GRP, _GRP)], src_v.at[buf], semis)
        d = pltpu.make_async_copy(
            dst_hbm.at[pl.ds(ebase + grp * _GRP, _GRP)], dst_v.at[buf], semid)
        return s, d

    def gather(buf, j, slot, sem):
        pltpu.async_copy(sup_hbm.at[src_v.at[buf, j]], rows_v.at[slot], sem)

    def gather_wait(slot, sem):
        pltpu.make_async_copy(sup_hbm.at[src_v.at[0, 0]], rows_v.at[slot],
                              sem).wait()

    for c in idx_copies(0, 0):
        c.start()
        c.wait()
    gather(0, 0, 0, sem0)

    @pl.loop(0, _NGRP // 2)
    def _pair(t):
        for buf in (0, 1):
            g = 2 * t + buf
            last = (buf == 1) & (t == _NGRP // 2 - 1)

            @pl.when(~last)
            def _():
                for c in idx_copies(g + 1, 1 - buf):
                    c.start()

            for j in range(_GRP - 1):
                gather(buf, j + 1, (j + 1) % 2, sems[(j + 1) % 2])
                gather_wait(j % 2, sems[j % 2])
                pltpu.sync_copy(rows_v.at[j % 2],
                                agg_sh.at[dst_v.at[buf, j]], add=True)

            @pl.when(~last)
            def _():
                for c in idx_copies(g + 1, 1 - buf):
                    c.wait()
                gather(1 - buf, 0, 0, sem0)

            gather_wait((_GRP - 1) % 2, sems[(_GRP - 1) % 2])
            pltpu.sync_copy(rows_v.at[(_GRP - 1) % 2],
                            agg_sh.at[dst_v.at[buf, _GRP - 1]], add=True)

    plsc.subcore_barrier()
    pltpu.sync_copy(agg_sh.at[pl.ds(zbase, _ZROWS)],
                    out_hbm.at[cid, pl.ds(zbase, _ZROWS)])


_sc_propagate = functools.partial(
    pl.kernel,
    out_type=jax.ShapeDtypeStruct((_NC, _ROWS, _D), jnp.float32),
    mesh=plsc.VectorSubcoreMesh(core_axis_name="c", subcore_axis_name="s",
                                num_cores=_NC, num_subcores=_NS),
    scratch_types=[
        pltpu.VMEM((2, _GRP, _CHUNK), jnp.int32),
        pltpu.VMEM((2, _GRP, _CHUNK), jnp.int32),
        pltpu.VMEM((2, _CHUNK, _D), jnp.float32),
        pltpu.VMEM_SHARED((_ROWS, _D), jnp.float32),
        pltpu.SemaphoreType.DMA,
        pltpu.SemaphoreType.DMA,
        pltpu.SemaphoreType.DMA,
        pltpu.SemaphoreType.DMA,
    ],
)(_sc_propagate_body)


def _tc_norm_mm_body(f_ref, w_ref, o_ref):
    x = f_ref[...]
    nrm = jnp.sqrt(jnp.sum(x * x, axis=1, keepdims=True))
    x = x / jnp.maximum(nrm, 1e-12)
    o_ref[...] = jnp.dot(x, w_ref[...], preferred_element_type=jnp.float32)


def _tc_combine_mm_body(p_ref, b_ref, w_ref, o_ref):
    h = jnp.maximum(p_ref[0, :_N] + p_ref[1, :_N] + b_ref[...], 0.0)
    o_ref[...] = jnp.dot(h, w_ref[...], preferred_element_type=jnp.float32)


def _tc_head_body(p_ref, b2_ref, y_ref, wy_ref, by_ref, wfc_ref, bfc_ref,
                  o_ref):
    h = p_ref[0, :_N] + p_ref[1, :_N] + b2_ref[...]
    y_emb = jnp.dot(y_ref[...], wy_ref[...],
                    preferred_element_type=jnp.float32) + by_ref[...]
    z = (jnp.dot(y_emb, wfc_ref[0], preferred_element_type=jnp.float32)
         + jnp.dot(h, wfc_ref[1], preferred_element_type=jnp.float32)
         + bfc_ref[...])
    m = jnp.max(z, axis=1, keepdims=True)
    e = jnp.exp(z - m)
    o_ref[...] = e / jnp.sum(e, axis=1, keepdims=True)


def kernel(features, edge_index, y, W1, b1, W2, b2, Wy, by, Wfc, bfc):
    pad = _EPAD - _E
    src = jnp.concatenate([edge_index[0], jnp.zeros((pad,), jnp.int32)])
    dst = jnp.concatenate([edge_index[1],
                           jnp.full((pad,), _N, jnp.int32)])
    src2d = src.reshape(_NW * _CPT, _CHUNK)
    dst2d = dst.reshape(_NW * _CPT, _CHUNK)

    support1 = pl.pallas_call(
        _tc_norm_mm_body,
        out_shape=jax.ShapeDtypeStruct((_N, _D), jnp.float32),
    )(features, W1)

    part1 = _sc_propagate(support1, src2d, dst2d)

    support2 = pl.pallas_call(
        _tc_combine_mm_body,
        out_shape=jax.ShapeDtypeStruct((_N, _D), jnp.float32),
    )(part1, b1.reshape(1, _D), W2)

    part2 = _sc_propagate(support2, src2d, dst2d)

    out = pl.pallas_call(
        _tc_head_body,
        out_shape=jax.ShapeDtypeStruct((_N, _NCLS), jnp.float32),
    )(part2, b2.reshape(1, _D), y, Wy, by.reshape(1, _D),
      Wfc.reshape(2, _D, _NCLS), bfc.reshape(1, _NCLS))

    return out


# speedup vs baseline: 4.1033x; 1.0507x over previous
import functools

import jax
import jax.numpy as jnp
from jax import lax
from jax.experimental import pallas as pl
from jax.experimental.pallas import tpu as pltpu
from jax.experimental.pallas import tpu_sc as plsc

_N = 10000
_E = 320000
_D = 128
_NCLS = 16

_NC = 2
_NS = 16
_NW = _NC * _NS
_CHUNK = 128
_CPT = 80
_GRP = 8
_NGRP = _CPT // _GRP
_EPAD = _NW * _CPT * _CHUNK
_ROWS = 10112
_ZROWS = _ROWS // _NS


def _sc_propagate_body(sup_hbm, src_hbm, dst_hbm, out_hbm,
                       src_v, dst_v, rows_v, agg_sh,
                       sem0, sem1, semis, semid):
    cid = lax.axis_index("c")
    sid = lax.axis_index("s")
    wid = cid * _NS + sid
    sems = (sem0, sem1)

    @pl.loop(0, _CHUNK)
    def _zero_buf(i):
        for j in range(_D // 16):
            rows_v[0, i, pl.ds(j * 16, 16)] = jnp.zeros((16,), jnp.float32)

    zbase = sid * _ZROWS
    off = 0
    for blk in (128, 128, 128, 128, _ZROWS - 4 * 128):
        pltpu.sync_copy(rows_v.at[0, pl.ds(0, blk)],
                        agg_sh.at[pl.ds(zbase + off, blk)])
        off += blk
    plsc.subcore_barrier()

    ebase = wid * _CPT

    def idx_copies(grp, buf):
        s = pltpu.make_async_copy(
            src_hbm.at[pl.ds(ebase + grp * _GRP, _GRP)], src_v.at[buf], semis)
        d = pltpu.make_async_copy(
            dst_hbm.at[pl.ds(ebase + grp * _GRP, _GRP)], dst_v.at[buf], semid)
        return s, d

    def gather(buf, j, slot, sem):
        pltpu.async_copy(sup_hbm.at[src_v.at[buf, j]], rows_v.at[slot], sem)

    def gather_wait(slot, sem):
        pltpu.make_async_copy(sup_hbm.at[src_v.at[0, 0]], rows_v.at[slot],
                              sem).wait()

    for c in idx_copies(0, 0):
        c.start()
        c.wait()
    gather(0, 0, 0, sem0)

    @pl.loop(0, _NGRP // 2)
    def _pair(t):
        for buf in (0, 1):
            g = 2 * t + buf
            last = (buf == 1) & (t == _NGRP // 2 - 1)

            @pl.when(~last)
            def _():
                for c in idx_copies(g + 1, 1 - buf):
                    c.start()

            for j in range(_GRP - 1):
                gather(buf, j + 1, (j + 1) % 2, sems[(j + 1) % 2])
                gather_wait(j % 2, sems[j % 2])
                pltpu.sync_copy(rows_v.at[j % 2],
                                agg_sh.at[dst_v.at[buf, j]], add=True)

            @pl.when(~last)
            def _():
                for c in idx_copies(g + 1, 1 - buf):
                    c.wait()
                gather(1 - buf, 0, 0, sem0)

            gather_wait((_GRP - 1) % 2, sems[(_GRP - 1) % 2])
            pltpu.sync_copy(rows_v.at[(_GRP - 1) % 2],
                            agg_sh.at[dst_v.at[buf, _GRP - 1]], add=True)

    plsc.subcore_barrier()
    pltpu.sync_copy(agg_sh.at[pl.ds(zbase, _ZROWS)],
                    out_hbm.at[cid, pl.ds(zbase, _ZROWS)])


_sc_propagate = functools.partial(
    pl.kernel,
    out_type=jax.ShapeDtypeStruct((_NC, _ROWS, _D), jnp.float32),
    mesh=plsc.VectorSubcoreMesh(core_axis_name="c", subcore_axis_name="s",
                                num_cores=_NC, num_subcores=_NS),
    scratch_types=[
        pltpu.VMEM((2, _GRP, _CHUNK), jnp.int32),
        pltpu.VMEM((2, _GRP, _CHUNK), jnp.int32),
        pltpu.VMEM((2, _CHUNK, _D), jnp.float32),
        pltpu.VMEM_SHARED((_ROWS, _D), jnp.float32),
        pltpu.SemaphoreType.DMA,
        pltpu.SemaphoreType.DMA,
        pltpu.SemaphoreType.DMA,
        pltpu.SemaphoreType.DMA,
    ],
)(_sc_propagate_body)


def _tc_norm_mm_body(f_ref, w_ref, o_ref):
    x = f_ref[...]
    nrm = jnp.sqrt(jnp.sum(x * x, axis=1, keepdims=True))
    x = x / jnp.maximum(nrm, 1e-12)
    o_ref[...] = jnp.dot(x, w_ref[...], preferred_element_type=jnp.float32)


def _tc_combine_mm_body(p_ref, b_ref, w_ref, o_ref):
    h = jnp.maximum(p_ref[0, :_N] + p_ref[1, :_N] + b_ref[...], 0.0)
    o_ref[...] = jnp.dot(h, w_ref[...], preferred_element_type=jnp.float32)


def _tc_head_body(p_ref, b2_ref, y_ref, wy_ref, by_ref, wfc_ref, bfc_ref,
                  o_ref):
    h = p_ref[0, :_N] + p_ref[1, :_N] + b2_ref[...]
    y_emb = jnp.dot(y_ref[...], wy_ref[...],
                    preferred_element_type=jnp.float32) + by_ref[...]
    z = (jnp.dot(y_emb, wfc_ref[0], preferred_element_type=jnp.float32)
         + jnp.dot(h, wfc_ref[1], preferred_element_type=jnp.float32)
         + bfc_ref[...])
    m = jnp.max(z, axis=1, keepdims=True)
    e = jnp.exp(z - m)
    o_ref[...] = e / jnp.sum(e, axis=1, keepdims=True)


def kernel(features, edge_index, y, W1, b1, W2, b2, Wy, by, Wfc, bfc):
    pad = _EPAD - _E
    trash = _N + (jnp.arange(pad, dtype=jnp.int32) % (_ROWS - _N))
    src = jnp.concatenate([edge_index[0], jnp.zeros((pad,), jnp.int32)])
    dst = jnp.concatenate([edge_index[1], trash])
    src2d = src.reshape(_NW * _CPT, _CHUNK)
    dst2d = dst.reshape(_NW * _CPT, _CHUNK)

    support1 = pl.pallas_call(
        _tc_norm_mm_body,
        out_shape=jax.ShapeDtypeStruct((_N, _D), jnp.float32),
    )(features, W1)

    part1 = _sc_propagate(support1, src2d, dst2d)

    support2 = pl.pallas_call(
        _tc_combine_mm_body,
        out_shape=jax.ShapeDtypeStruct((_N, _D), jnp.float32),
    )(part1, b1.reshape(1, _D), W2)

    part2 = _sc_propagate(support2, src2d, dst2d)

    out = pl.pallas_call(
        _tc_head_body,
        out_shape=jax.ShapeDtypeStruct((_N, _NCLS), jnp.float32),
    )(part2, b2.reshape(1, _D), y, Wy, by.reshape(1, _D),
      Wfc.reshape(2, _D, _NCLS), bfc.reshape(1, _NCLS))

    return out
